# baseline (speedup 1.0000x reference)
import functools
import math

import jax
import jax.numpy as jnp
from jax import lax
from jax.experimental import pallas as pl
from jax.experimental.pallas import tpu as pltpu
from jax.experimental.pallas import tpu_sc as plsc

N = 10000
E = 320000
G = 256
D = 178
DP = 192
NC = 2
NS = 16
NW = NC * NS
ET = E // NW
EC = 48
NCH = 210
NP = 10240
NPH = NP // 2
RPS = NPH // NS
RB = 1000
NRB = N // RB

_INV_SQRT_D = 1.0 / math.sqrt(float(D))


def _proj_body(x_ref, wq_ref, bq_ref, wkv_ref, bkv_ref, ws_ref, bs_ref,
               q_ref, kv_ref, s_ref):
    h = x_ref[...]
    q_ref[...] = jnp.dot(h, wq_ref[...], preferred_element_type=jnp.float32, precision=lax.Precision.HIGHEST) + bq_ref[...]
    kv_ref[...] = jnp.dot(h, wkv_ref[...], preferred_element_type=jnp.float32, precision=lax.Precision.HIGHEST) + bkv_ref[...]
    s_ref[...] = jnp.dot(h, ws_ref[...], preferred_element_type=jnp.float32, precision=lax.Precision.HIGHEST) + bs_ref[...]


def _proj(x, wq, bq, wkv, bkv, ws, bs):
    full = lambda shape: pl.BlockSpec(shape, lambda i: (0,) * len(shape))
    return pl.pallas_call(
        _proj_body,
        grid=(NRB,),
        in_specs=[
            pl.BlockSpec((RB, D), lambda i: (i, 0)),
            full((D, DP)), full((1, DP)),
            full((D, 2 * DP)), full((1, 2 * DP)),
            full((D, D)), full((1, D)),
        ],
        out_specs=[
            pl.BlockSpec((RB, DP), lambda i: (i, 0)),
            pl.BlockSpec((RB, 2 * DP), lambda i: (i, 0)),
            pl.BlockSpec((RB, D), lambda i: (i, 0)),
        ],
        out_shape=[
            jax.ShapeDtypeStruct((N, DP), jnp.float32),
            jax.ShapeDtypeStruct((N, 2 * DP), jnp.float32),
            jax.ShapeDtypeStruct((N, D), jnp.float32),
        ],
    )(x, wq, bq, wkv, bkv, ws, bs)


def _combine_proj_body(agg_ref, s_ref, wq_ref, bq_ref, wkv_ref, bkv_ref,
                       ws_ref, bs_ref, q_ref, kv_ref, s_out_ref):
    aggsum = agg_ref[0] + agg_ref[1]
    den = jnp.maximum(aggsum[:, D:D + 1], 1e-16)
    h = jax.nn.relu(aggsum[:, :D] / den + s_ref[...])
    q_ref[...] = jnp.dot(h, wq_ref[...], preferred_element_type=jnp.float32, precision=lax.Precision.HIGHEST) + bq_ref[...]
    kv_ref[...] = jnp.dot(h, wkv_ref[...], preferred_element_type=jnp.float32, precision=lax.Precision.HIGHEST) + bkv_ref[...]
    s_out_ref[...] = jnp.dot(h, ws_ref[...], preferred_element_type=jnp.float32, precision=lax.Precision.HIGHEST) + bs_ref[...]


def _combine_proj(agg, s_prev, wq, bq, wkv, bkv, ws, bs):
    full = lambda shape: pl.BlockSpec(shape, lambda i: (0,) * len(shape))
    return pl.pallas_call(
        _combine_proj_body,
        grid=(NRB,),
        in_specs=[
            pl.BlockSpec((NC, RB, DP), lambda i: (0, i, 0)),
            pl.BlockSpec((RB, D), lambda i: (i, 0)),
            full((D, DP)), full((1, DP)),
            full((D, 2 * DP)), full((1, 2 * DP)),
            full((D, D)), full((1, D)),
        ],
        out_specs=[
            pl.BlockSpec((RB, DP), lambda i: (i, 0)),
            pl.BlockSpec((RB, 2 * DP), lambda i: (i, 0)),
            pl.BlockSpec((RB, D), lambda i: (i, 0)),
        ],
        out_shape=[
            jax.ShapeDtypeStruct((N, DP), jnp.float32),
            jax.ShapeDtypeStruct((N, 2 * DP), jnp.float32),
            jax.ShapeDtypeStruct((N, D), jnp.float32),
        ],
    )(agg, s_prev, wq, bq, wkv, bkv, ws, bs)


@functools.lru_cache(maxsize=None)
def _make_edge_sc():
    mesh = plsc.VectorSubcoreMesh(
        core_axis_name="c", subcore_axis_name="s",
        num_cores=NC, num_subcores=NS)
    return functools.partial(
        pl.kernel,
        out_type=jax.ShapeDtypeStruct((NC, NP, DP), jnp.float32),
        mesh=mesh,
        compiler_params=pltpu.CompilerParams(
            needs_layout_passes=False, use_tc_tiling_on_sc=False),
        scratch_types=[
            pltpu.VMEM_SHARED((NPH + 16, DP), jnp.float32),
            [pltpu.VMEM((EC,), jnp.int32) for _ in range(2)],
            [pltpu.VMEM((EC,), jnp.int32) for _ in range(2)],
            [pltpu.VMEM((EC,), jnp.int32) for _ in range(2)],
            [pltpu.VMEM((EC, DP), jnp.float32) for _ in range(2)],
            [pltpu.VMEM((EC, 2 * DP), jnp.float32) for _ in range(2)],
            pltpu.VMEM((EC, DP), jnp.float32),
            [pltpu.SemaphoreType.DMA for _ in range(2)],
            [pltpu.SemaphoreType.DMA for _ in range(2)],
            [pltpu.SemaphoreType.DMA for _ in range(2)],
        ],
    )(_edge_sc_body)


def _edge_sc(q, kv, src, dst):
    return _make_edge_sc()(q, kv, src, dst)


def _edge_sc_body(q_hbm, kv_hbm, src_hbm, dst_hbm, out_hbm,
                  agg_sh, src_v, dst_v, rel_v, q_rows, kv_rows, out_rows,
                  sem_i, sem_q, sem_kv):
    c = lax.axis_index("c")
    s = lax.axis_index("s")
    wid = c * NS + s
    ebase = wid * ET
    rbase = s * RPS

    def _issue_idx(j, b):
        base = ebase + j * EC
        pltpu.async_copy(src_hbm.at[pl.ds(base, EC)], src_v[b], sem_i[b])
        pltpu.async_copy(dst_hbm.at[pl.ds(base, EC)], dst_v[b], sem_i[b])

    def _wait_idx(b):
        pltpu.make_async_copy(src_hbm.at[pl.ds(0, EC)], src_v[b], sem_i[b]).wait()
        pltpu.make_async_copy(dst_hbm.at[pl.ds(0, EC)], dst_v[b], sem_i[b]).wait()

    def _issue_gather(b):
        pltpu.async_copy(q_hbm.at[dst_v[b]], q_rows[b], sem_q[b])
        pltpu.async_copy(kv_hbm.at[src_v[b]], kv_rows[b], sem_kv[b])

    def _wait_gather(b):
        pltpu.make_async_copy(q_hbm.at[dst_v[b]], q_rows[b], sem_q[b]).wait()
        pltpu.make_async_copy(kv_hbm.at[src_v[b]], kv_rows[b], sem_kv[b]).wait()

    for p in range(2):
        lo = p * NPH

        def _zero_body(e, _):
            zero16 = jnp.zeros((16,), jnp.float32)
            for j in range(DP // 16):
                out_rows[e, pl.ds(16 * j, 16)] = zero16
            return 0
        lax.fori_loop(0, EC, _zero_body, 0)
        for t in range(RPS // EC):
            pltpu.sync_copy(out_rows, agg_sh.at[pl.ds(rbase + t * EC, EC), :])
        pltpu.sync_copy(out_rows.at[pl.ds(0, RPS % EC), :],
                        agg_sh.at[pl.ds(rbase + (RPS // EC) * EC, RPS % EC), :])
        plsc.subcore_barrier()

        def _compute_rel(j, b):
            for t in range(EC // 16):
                d16 = dst_v[b][pl.ds(16 * t, 16)]
                rel = d16 - lo
                eid = j * EC + 16 * t + lax.iota(jnp.int32, 16)
                ok = (rel >= 0) & (rel < NPH) & (eid < ET)
                rel_v[b][pl.ds(16 * t, 16)] = jnp.where(ok, rel, NPH)

        _issue_idx(0, 0)
        _wait_idx(0)
        _compute_rel(0, 0)
        _issue_gather(0)
        _issue_idx(1, 1)

        def _chunk_body(i2, _):
            for b in range(2):
                j = 2 * i2 + b
                _wait_gather(b)

                @pl.when(j + 1 < NCH)
                def _():
                    _wait_idx(1 - b)
                    _compute_rel(j + 1, 1 - b)
                    _issue_gather(1 - b)

                @plsc.parallel_loop(0, EC, 1, unroll=8)
                def _edge_body(e):
                    acc = jnp.zeros((16,), jnp.float32)
                    for jj in range(DP // 16):
                        acc = acc + q_rows[b][e, pl.ds(16 * jj, 16)] * kv_rows[b][e, pl.ds(16 * jj, 16)]
                    alpha = jnp.sum(acc) * _INV_SQRT_D
                    ex = jnp.exp(jnp.full((16,), alpha, jnp.float32))
                    for jj in range(DP // 16):
                        out_rows[e, pl.ds(16 * jj, 16)] = kv_rows[b][e, pl.ds(DP + 16 * jj, 16)] * ex
                pltpu.sync_copy(out_rows, agg_sh.at[rel_v[b]], add=True)

                @pl.when(j + 2 < NCH)
                def _():
                    _issue_idx(j + 2, b)
            return 0

        lax.fori_loop(0, NCH // 2, _chunk_body, 0)
        plsc.subcore_barrier()

        pltpu.sync_copy(agg_sh.at[pl.ds(rbase, RPS), :],
                        out_hbm.at[c, pl.ds(lo + rbase, RPS), :])


def _final_body(agg_ref, s_ref, batch_ref, gf_ref,
                gfw1_ref, gfb1_ref, gfw2_ref, gfb2_ref,
                w1a_ref, w1b_ref, b1_ref, w2_ref, b2_ref,
                w3_ref, b3_ref, w4_ref, b4_ref,
                out_ref, pooled_s, cnt_s):
    i = pl.program_id(0)
    aggsum = agg_ref[0] + agg_ref[1]
    den = jnp.maximum(aggsum[:, D:D + 1], 1e-16)
    h = jax.nn.relu(aggsum[:, :D] / den + s_ref[...])

    batch = batch_ref[0, 0, :]
    gids = lax.broadcasted_iota(jnp.int32, (G, RB), 0)
    m = (batch[None, :] == gids).astype(jnp.float32)
    pool_blk = jnp.dot(m, h, preferred_element_type=jnp.float32, precision=lax.Precision.HIGHEST)
    cnt_blk = jnp.sum(m, axis=1, keepdims=True)

    @pl.when(i == 0)
    def _():
        pooled_s[...] = jnp.zeros_like(pooled_s)
        cnt_s[...] = jnp.zeros_like(cnt_s)

    pooled_s[...] += pool_blk
    cnt_s[...] += cnt_blk

    @pl.when(i == NRB - 1)
    def _():
        pm = pooled_s[...] / jnp.maximum(cnt_s[...], 1.0)
        g1 = jax.nn.relu(jnp.dot(gf_ref[...], gfw1_ref[...],
                                 preferred_element_type=jnp.float32, precision=lax.Precision.HIGHEST) + gfb1_ref[...])
        g2 = jax.nn.relu(jnp.dot(g1, gfw2_ref[...],
                                 preferred_element_type=jnp.float32, precision=lax.Precision.HIGHEST) + gfb2_ref[...])
        z1 = jax.nn.relu(jnp.dot(pm, w1a_ref[...], preferred_element_type=jnp.float32, precision=lax.Precision.HIGHEST)
                         + jnp.dot(g2, w1b_ref[...], preferred_element_type=jnp.float32, precision=lax.Precision.HIGHEST)
                         + b1_ref[...])
        z2 = jax.nn.relu(jnp.dot(z1, w2_ref[...], preferred_element_type=jnp.float32, precision=lax.Precision.HIGHEST) + b2_ref[...])
        z3 = jax.nn.relu(jnp.dot(z2, w3_ref[...], preferred_element_type=jnp.float32, precision=lax.Precision.HIGHEST) + b3_ref[...])
        out_ref[...] = jnp.dot(z3, w4_ref[...], preferred_element_type=jnp.float32, precision=lax.Precision.HIGHEST) + b4_ref[...]


def _final(agg, s_prev, batch3d, gf, gfw1, gfb1, gfw2, gfb2,
           w1a, w1b, b1, w2, b2, w3, b3, w4, b4):
    full = lambda shape: pl.BlockSpec(shape, lambda i: (0,) * len(shape))
    return pl.pallas_call(
        _final_body,
        grid=(NRB,),
        in_specs=[
            pl.BlockSpec((NC, RB, DP), lambda i: (0, i, 0)),
            pl.BlockSpec((RB, D), lambda i: (i, 0)),
            pl.BlockSpec((1, 1, RB), lambda i: (i, 0, 0)),
            full((G, 41)),
            full((41, 64)), full((1, 64)), full((64, 64)), full((1, 64)),
            full((D, 512)), full((64, 512)), full((1, 512)),
            full((512, 512)), full((1, 512)),
            full((512, 128)), full((1, 128)),
            full((128, 1)), full((1, 1)),
        ],
        out_specs=pl.BlockSpec((G, 1), lambda i: (0, 0)),
        out_shape=jax.ShapeDtypeStruct((G, 1), jnp.float32),
        scratch_shapes=[
            pltpu.VMEM((G, D), jnp.float32),
            pltpu.VMEM((G, 1), jnp.float32),
        ],
    )(agg, s_prev, batch3d, gf, gfw1, gfb1, gfw2, gfb2,
      w1a, w1b, b1, w2, b2, w3, b3, w4, b4)


def _pack_layer(wq, bq, wk, bk, wv, bv):
    wq_p = jnp.pad(wq, ((0, 0), (0, DP - D)))
    bq_p = jnp.pad(bq, (0, DP - D)).reshape(1, DP)
    wk_p = jnp.pad(wk, ((0, 0), (0, DP - D)))
    wv_p = jnp.pad(wv, ((0, 0), (0, DP - D)))
    wkv = jnp.concatenate([wk_p, wv_p], axis=1)
    bk_p = jnp.pad(bk, (0, DP - D))
    bv_p = jnp.pad(bv, (0, DP - D)).at[D].set(1.0)
    bkv = jnp.concatenate([bk_p, bv_p]).reshape(1, 2 * DP)
    return wq_p, bq_p, wkv, bkv


def kernel(x, edge_index, global_features, batch,
           Wq0, bq0, Wk0, bk0, Wv0, bv0, Ws0, bs0,
           Wq1, bq1, Wk1, bk1, Wv1, bv1, Ws1, bs1,
           gfW1, gfb1, gfW2, gfb2,
           W1, b1, W2, b2, W3, b3, W4, b4):
    x = x.astype(jnp.float32)
    pad = jnp.zeros((2, NCH * EC * NW - E + 64), jnp.int32)
    eip = jnp.concatenate([edge_index, pad], axis=1)
    src = eip[0]
    dst = eip[1]

    wq0, bq0p, wkv0, bkv0 = _pack_layer(Wq0, bq0, Wk0, bk0, Wv0, bv0)
    wq1, bq1p, wkv1, bkv1 = _pack_layer(Wq1, bq1, Wk1, bk1, Wv1, bv1)

    q0, kv0, s0 = _proj(x, wq0, bq0p, wkv0, bkv0, Ws0, bs0.reshape(1, D))
    agg0 = _edge_sc(q0, kv0, src, dst)
    q1, kv1, s1 = _combine_proj(agg0, s0, wq1, bq1p, wkv1, bkv1,
                                Ws1, bs1.reshape(1, D))
    agg1 = _edge_sc(q1, kv1, src, dst)

    out = _final(agg1, s1, batch.reshape(NRB, 1, RB), global_features,
                 gfW1, gfb1.reshape(1, 64), gfW2, gfb2.reshape(1, 64),
                 W1[:D], W1[D:], b1.reshape(1, 512),
                 W2, b2.reshape(1, 512), W3, b3.reshape(1, 128),
                 W4, b4.reshape(1, 1))
    return jnp.squeeze(out, axis=-1)


# speedup vs baseline: 1.5341x; 1.5341x over previous
import functools
import math

import jax
import jax.numpy as jnp
from jax import lax
from jax.experimental import pallas as pl
from jax.experimental.pallas import tpu as pltpu
from jax.experimental.pallas import tpu_sc as plsc

N = 10000
E = 320000
G = 256
D = 178
DP = 192
NC = 2
NS = 16
NW = NC * NS
ET = E // NW
EC = 48
CAP = 10240
NP = 10240
SENT = 2 * NP
NPH = NP // 2
RPS = NPH // NS
RB = 1000
NRB = N // RB

_INV_SQRT_D = 1.0 / math.sqrt(float(D))


def _proj_body(x_ref, wq_ref, bq_ref, wkv_ref, bkv_ref, ws_ref, bs_ref,
               q_ref, kv_ref, s_ref):
    h = x_ref[...]
    q_ref[...] = jnp.dot(h, wq_ref[...], preferred_element_type=jnp.float32) + bq_ref[...]
    kv_ref[...] = jnp.dot(h, wkv_ref[...], preferred_element_type=jnp.float32) + bkv_ref[...]
    s_ref[...] = jnp.dot(h, ws_ref[...], preferred_element_type=jnp.float32) + bs_ref[...]


def _proj(x, wq, bq, wkv, bkv, ws, bs):
    full = lambda shape: pl.BlockSpec(shape, lambda i: (0,) * len(shape))
    return pl.pallas_call(
        _proj_body,
        grid=(NRB,),
        in_specs=[
            pl.BlockSpec((RB, D), lambda i: (i, 0)),
            full((D, DP)), full((1, DP)),
            full((D, 2 * DP)), full((1, 2 * DP)),
            full((D, D)), full((1, D)),
        ],
        out_specs=[
            pl.BlockSpec((RB, DP), lambda i: (i, 0)),
            pl.BlockSpec((RB, 2 * DP), lambda i: (i, 0)),
            pl.BlockSpec((RB, D), lambda i: (i, 0)),
        ],
        out_shape=[
            jax.ShapeDtypeStruct((N, DP), jnp.float32),
            jax.ShapeDtypeStruct((N, 2 * DP), jnp.float32),
            jax.ShapeDtypeStruct((N, D), jnp.float32),
        ],
    )(x, wq, bq, wkv, bkv, ws, bs)


def _combine_proj_body(agg_ref, s_ref, wq_ref, bq_ref, wkv_ref, bkv_ref,
                       ws_ref, bs_ref, q_ref, kv_ref, s_out_ref):
    aggsum = agg_ref[0] + agg_ref[1]
    den = jnp.maximum(aggsum[:, D:D + 1], 1e-16)
    h = jax.nn.relu(aggsum[:, :D] / den + s_ref[...])
    q_ref[...] = jnp.dot(h, wq_ref[...], preferred_element_type=jnp.float32) + bq_ref[...]
    kv_ref[...] = jnp.dot(h, wkv_ref[...], preferred_element_type=jnp.float32) + bkv_ref[...]
    s_out_ref[...] = jnp.dot(h, ws_ref[...], preferred_element_type=jnp.float32) + bs_ref[...]


def _combine_proj(agg, s_prev, wq, bq, wkv, bkv, ws, bs):
    full = lambda shape: pl.BlockSpec(shape, lambda i: (0,) * len(shape))
    return pl.pallas_call(
        _combine_proj_body,
        grid=(NRB,),
        in_specs=[
            pl.BlockSpec((NC, RB, DP), lambda i: (0, i, 0)),
            pl.BlockSpec((RB, D), lambda i: (i, 0)),
            full((D, DP)), full((1, DP)),
            full((D, 2 * DP)), full((1, 2 * DP)),
            full((D, D)), full((1, D)),
        ],
        out_specs=[
            pl.BlockSpec((RB, DP), lambda i: (i, 0)),
            pl.BlockSpec((RB, 2 * DP), lambda i: (i, 0)),
            pl.BlockSpec((RB, D), lambda i: (i, 0)),
        ],
        out_shape=[
            jax.ShapeDtypeStruct((N, DP), jnp.float32),
            jax.ShapeDtypeStruct((N, 2 * DP), jnp.float32),
            jax.ShapeDtypeStruct((N, D), jnp.float32),
        ],
    )(agg, s_prev, wq, bq, wkv, bkv, ws, bs)


@functools.lru_cache(maxsize=None)
def _make_partition_sc():
    mesh = plsc.VectorSubcoreMesh(
        core_axis_name="c", subcore_axis_name="s",
        num_cores=NC, num_subcores=NS)
    return functools.partial(
        pl.kernel,
        out_type=[
            jax.ShapeDtypeStruct((NW, 2, CAP), jnp.int32),
            jax.ShapeDtypeStruct((NW, 2, CAP), jnp.int32),
            jax.ShapeDtypeStruct((NW, 16), jnp.int32),
        ],
        mesh=mesh,
        compiler_params=pltpu.CompilerParams(
            needs_layout_passes=False, use_tc_tiling_on_sc=False),
        scratch_types=[
            pltpu.VMEM((ET,), jnp.int32),
            pltpu.VMEM((ET,), jnp.int32),
            [pltpu.VMEM((CAP + 16,), jnp.int32) for _ in range(2)],
            [pltpu.VMEM((CAP + 16,), jnp.int32) for _ in range(2)],
            pltpu.VMEM((16,), jnp.int32),
        ],
    )(_partition_sc_body)


def _partition_sc(src, dst):
    return _make_partition_sc()(src, dst)


def _partition_sc_body(src_hbm, dst_hbm, psrc_hbm, pdst_hbm, cnt_hbm,
                       in_src, in_dst, b_src, b_dst, cnt_v):
    c = lax.axis_index("c")
    s = lax.axis_index("s")
    wid = c * NS + s
    ebase = wid * ET

    pltpu.sync_copy(src_hbm.at[pl.ds(ebase, ET)], in_src)
    pltpu.sync_copy(dst_hbm.at[pl.ds(ebase, ET)], in_dst)

    zero16 = jnp.zeros((16,), jnp.int32)
    sent16 = jnp.full((16,), SENT, jnp.int32)

    @plsc.parallel_loop(0, (CAP + 16) // 16, 1, unroll=8)
    def _prefill(t):
        for h in range(2):
            b_src[h][pl.ds(16 * t, 16)] = zero16
            b_dst[h][pl.ds(16 * t, 16)] = sent16

    def _compact(t, offs):
        lo_off, hi_off = offs
        s16 = in_src[pl.ds(16 * t, 16)]
        d16 = in_dst[pl.ds(16 * t, 16)]
        m_lo = d16 < NPH
        plsc.store_compressed(b_src[0].at[pl.ds(lo_off, 16)], s16, mask=m_lo)
        plsc.store_compressed(b_dst[0].at[pl.ds(lo_off, 16)], d16, mask=m_lo)
        m_hi = jnp.logical_not(m_lo)
        plsc.store_compressed(b_src[1].at[pl.ds(hi_off, 16)], s16, mask=m_hi)
        plsc.store_compressed(b_dst[1].at[pl.ds(hi_off, 16)], d16, mask=m_hi)
        nlo = jnp.sum(m_lo.astype(jnp.int32))
        return lo_off + nlo, hi_off + (16 - nlo)

    lo_n, hi_n = lax.fori_loop(0, ET // 16, _compact,
                               (jnp.int32(0), jnp.int32(0)))

    iota16 = lax.iota(jnp.int32, 16)
    cnt_v[...] = jnp.where(iota16 == 0, lo_n,
                           jnp.where(iota16 == 1, hi_n, 0))
    for h in range(2):
        pltpu.sync_copy(b_src[h].at[pl.ds(0, CAP)], psrc_hbm.at[wid, h, :])
        pltpu.sync_copy(b_dst[h].at[pl.ds(0, CAP)], pdst_hbm.at[wid, h, :])
    pltpu.sync_copy(cnt_v, cnt_hbm.at[wid])


@functools.lru_cache(maxsize=None)
def _make_edge_sc():
    mesh = plsc.VectorSubcoreMesh(
        core_axis_name="c", subcore_axis_name="s",
        num_cores=NC, num_subcores=NS)
    return functools.partial(
        pl.kernel,
        out_type=jax.ShapeDtypeStruct((NC, NP, DP), jnp.float32),
        mesh=mesh,
        compiler_params=pltpu.CompilerParams(
            needs_layout_passes=False, use_tc_tiling_on_sc=False),
        scratch_types=[
            pltpu.VMEM_SHARED((NPH + 16, DP), jnp.float32),
            pltpu.VMEM((16,), jnp.int32),
            [pltpu.VMEM((EC,), jnp.int32) for _ in range(2)],
            [pltpu.VMEM((EC,), jnp.int32) for _ in range(2)],
            [pltpu.VMEM((EC,), jnp.int32) for _ in range(2)],
            [pltpu.VMEM((EC, DP), jnp.float32) for _ in range(2)],
            [pltpu.VMEM((EC, 2 * DP), jnp.float32) for _ in range(2)],
            pltpu.VMEM((EC, DP), jnp.float32),
            [pltpu.SemaphoreType.DMA for _ in range(2)],
            [pltpu.SemaphoreType.DMA for _ in range(2)],
            [pltpu.SemaphoreType.DMA for _ in range(2)],
        ],
    )(_edge_sc_body)


def _edge_sc(q, kv, psrc, pdst, cnt):
    return _make_edge_sc()(q, kv, psrc, pdst, cnt)


def _edge_sc_body(q_hbm, kv_hbm, psrc_hbm, pdst_hbm, cnt_hbm, out_hbm,
                  agg_sh, cnt_v, src_v, dst_v, rel_v, q_rows, kv_rows,
                  out_rows, sem_i, sem_q, sem_kv):
    c = lax.axis_index("c")
    s = lax.axis_index("s")
    wid = c * NS + s
    rbase = s * RPS

    pltpu.sync_copy(cnt_hbm.at[wid], cnt_v)

    for p in range(2):
        lo = p * NPH
        n_edges = cnt_v[pl.ds(0, 16)][p]
        nch = jnp.maximum(2 * ((n_edges + 2 * EC - 1) // (2 * EC)), 2)

        def _issue_idx(j, b):
            pltpu.async_copy(psrc_hbm.at[wid, p, pl.ds(j * EC, EC)], src_v[b], sem_i[b])
            pltpu.async_copy(pdst_hbm.at[wid, p, pl.ds(j * EC, EC)], dst_v[b], sem_i[b])

        def _wait_idx(b):
            pltpu.make_async_copy(psrc_hbm.at[wid, p, pl.ds(0, EC)], src_v[b], sem_i[b]).wait()
            pltpu.make_async_copy(pdst_hbm.at[wid, p, pl.ds(0, EC)], dst_v[b], sem_i[b]).wait()

        def _issue_gather(b):
            pltpu.async_copy(q_hbm.at[dst_v[b]], q_rows[b], sem_q[b])
            pltpu.async_copy(kv_hbm.at[src_v[b]], kv_rows[b], sem_kv[b])

        def _wait_gather(b):
            pltpu.make_async_copy(q_hbm.at[dst_v[b]], q_rows[b], sem_q[b]).wait()
            pltpu.make_async_copy(kv_hbm.at[src_v[b]], kv_rows[b], sem_kv[b]).wait()

        def _zero_body(e, _):
            zero16 = jnp.zeros((16,), jnp.float32)
            for j in range(DP // 16):
                out_rows[e, pl.ds(16 * j, 16)] = zero16
            return 0
        lax.fori_loop(0, EC, _zero_body, 0)
        for t in range(RPS // EC):
            pltpu.sync_copy(out_rows, agg_sh.at[pl.ds(rbase + t * EC, EC), :])
        pltpu.sync_copy(out_rows.at[pl.ds(0, RPS % EC), :],
                        agg_sh.at[pl.ds(rbase + (RPS // EC) * EC, RPS % EC), :])
        plsc.subcore_barrier()

        def _compute_rel(b):
            for t in range(EC // 16):
                d16 = dst_v[b][pl.ds(16 * t, 16)]
                rel = d16 - lo
                ok = (rel >= 0) & (rel < NPH)
                rel_v[b][pl.ds(16 * t, 16)] = jnp.where(ok, rel, NPH)

        _issue_idx(0, 0)
        _wait_idx(0)
        _compute_rel(0)
        _issue_gather(0)
        _issue_idx(1, 1)

        def _chunk_body(i2, _):
            for b in range(2):
                j = 2 * i2 + b
                _wait_gather(b)

                @pl.when(j + 1 < nch)
                def _():
                    _wait_idx(1 - b)
                    _compute_rel(1 - b)
                    _issue_gather(1 - b)

                @plsc.parallel_loop(0, EC, 1, unroll=8)
                def _edge_body(e):
                    acc = jnp.zeros((16,), jnp.float32)
                    for jj in range(DP // 16):
                        acc = acc + q_rows[b][e, pl.ds(16 * jj, 16)] * kv_rows[b][e, pl.ds(16 * jj, 16)]
                    alpha = jnp.sum(acc) * _INV_SQRT_D
                    ex = jnp.exp(jnp.full((16,), alpha, jnp.float32))
                    for jj in range(DP // 16):
                        out_rows[e, pl.ds(16 * jj, 16)] = kv_rows[b][e, pl.ds(DP + 16 * jj, 16)] * ex

                pltpu.sync_copy(out_rows, agg_sh.at[rel_v[b]], add=True)

                @pl.when(j + 2 < nch)
                def _():
                    _issue_idx(j + 2, b)
            return 0

        lax.fori_loop(0, nch // 2, _chunk_body, 0)
        plsc.subcore_barrier()

        pltpu.sync_copy(agg_sh.at[pl.ds(rbase, RPS), :],
                        out_hbm.at[c, pl.ds(lo + rbase, RPS), :])


def _final_body(agg_ref, s_ref, batch_ref, gf_ref,
                gfw1_ref, gfb1_ref, gfw2_ref, gfb2_ref,
                w1a_ref, w1b_ref, b1_ref, w2_ref, b2_ref,
                w3_ref, b3_ref, w4_ref, b4_ref,
                out_ref, pooled_s, cnt_s):
    i = pl.program_id(0)
    aggsum = agg_ref[0] + agg_ref[1]
    den = jnp.maximum(aggsum[:, D:D + 1], 1e-16)
    h = jax.nn.relu(aggsum[:, :D] / den + s_ref[...])

    batch = batch_ref[0, 0, :]
    gids = lax.broadcasted_iota(jnp.int32, (G, RB), 0)
    m = (batch[None, :] == gids).astype(jnp.float32)
    pool_blk = jnp.dot(m, h, preferred_element_type=jnp.float32)
    cnt_blk = jnp.sum(m, axis=1, keepdims=True)

    @pl.when(i == 0)
    def _():
        pooled_s[...] = jnp.zeros_like(pooled_s)
        cnt_s[...] = jnp.zeros_like(cnt_s)

    pooled_s[...] += pool_blk
    cnt_s[...] += cnt_blk

    @pl.when(i == NRB - 1)
    def _():
        pm = pooled_s[...] / jnp.maximum(cnt_s[...], 1.0)
        g1 = jax.nn.relu(jnp.dot(gf_ref[...], gfw1_ref[...],
                                 preferred_element_type=jnp.float32) + gfb1_ref[...])
        g2 = jax.nn.relu(jnp.dot(g1, gfw2_ref[...],
                                 preferred_element_type=jnp.float32) + gfb2_ref[...])
        z1 = jax.nn.relu(jnp.dot(pm, w1a_ref[...], preferred_element_type=jnp.float32)
                         + jnp.dot(g2, w1b_ref[...], preferred_element_type=jnp.float32)
                         + b1_ref[...])
        z2 = jax.nn.relu(jnp.dot(z1, w2_ref[...], preferred_element_type=jnp.float32) + b2_ref[...])
        z3 = jax.nn.relu(jnp.dot(z2, w3_ref[...], preferred_element_type=jnp.float32) + b3_ref[...])
        out_ref[...] = jnp.dot(z3, w4_ref[...], preferred_element_type=jnp.float32) + b4_ref[...]


def _final(agg, s_prev, batch3d, gf, gfw1, gfb1, gfw2, gfb2,
           w1a, w1b, b1, w2, b2, w3, b3, w4, b4):
    full = lambda shape: pl.BlockSpec(shape, lambda i: (0,) * len(shape))
    return pl.pallas_call(
        _final_body,
        grid=(NRB,),
        in_specs=[
            pl.BlockSpec((NC, RB, DP), lambda i: (0, i, 0)),
            pl.BlockSpec((RB, D), lambda i: (i, 0)),
            pl.BlockSpec((1, 1, RB), lambda i: (i, 0, 0)),
            full((G, 41)),
            full((41, 64)), full((1, 64)), full((64, 64)), full((1, 64)),
            full((D, 512)), full((64, 512)), full((1, 512)),
            full((512, 512)), full((1, 512)),
            full((512, 128)), full((1, 128)),
            full((128, 1)), full((1, 1)),
        ],
        out_specs=pl.BlockSpec((G, 1), lambda i: (0, 0)),
        out_shape=jax.ShapeDtypeStruct((G, 1), jnp.float32),
        scratch_shapes=[
            pltpu.VMEM((G, D), jnp.float32),
            pltpu.VMEM((G, 1), jnp.float32),
        ],
    )(agg, s_prev, batch3d, gf, gfw1, gfb1, gfw2, gfb2,
      w1a, w1b, b1, w2, b2, w3, b3, w4, b4)


def _pack_layer(wq, bq, wk, bk, wv, bv):
    wq_p = jnp.pad(wq, ((0, 0), (0, DP - D)))
    bq_p = jnp.pad(bq, (0, DP - D)).reshape(1, DP)
    wk_p = jnp.pad(wk, ((0, 0), (0, DP - D)))
    wv_p = jnp.pad(wv, ((0, 0), (0, DP - D)))
    wkv = jnp.concatenate([wk_p, wv_p], axis=1)
    bk_p = jnp.pad(bk, (0, DP - D))
    bv_p = jnp.pad(bv, (0, DP - D)).at[D].set(1.0)
    bkv = jnp.concatenate([bk_p, bv_p]).reshape(1, 2 * DP)
    return wq_p, bq_p, wkv, bkv


def kernel(x, edge_index, global_features, batch,
           Wq0, bq0, Wk0, bk0, Wv0, bv0, Ws0, bs0,
           Wq1, bq1, Wk1, bk1, Wv1, bv1, Ws1, bs1,
           gfW1, gfb1, gfW2, gfb2,
           W1, b1, W2, b2, W3, b3, W4, b4):
    x = x.astype(jnp.float32)
    src = edge_index[0]
    dst = edge_index[1]

    wq0, bq0p, wkv0, bkv0 = _pack_layer(Wq0, bq0, Wk0, bk0, Wv0, bv0)
    wq1, bq1p, wkv1, bkv1 = _pack_layer(Wq1, bq1, Wk1, bk1, Wv1, bv1)

    psrc, pdst, cnt = _partition_sc(src, dst)
    q0, kv0, s0 = _proj(x, wq0, bq0p, wkv0, bkv0, Ws0, bs0.reshape(1, D))
    agg0 = _edge_sc(q0, kv0, psrc, pdst, cnt)
    q1, kv1, s1 = _combine_proj(agg0, s0, wq1, bq1p, wkv1, bkv1,
                                Ws1, bs1.reshape(1, D))
    agg1 = _edge_sc(q1, kv1, psrc, pdst, cnt)

    out = _final(agg1, s1, batch.reshape(NRB, 1, RB), global_features,
                 gfW1, gfb1.reshape(1, 64), gfW2, gfb2.reshape(1, 64),
                 W1[:D], W1[D:], b1.reshape(1, 512),
                 W2, b2.reshape(1, 512), W3, b3.reshape(1, 128),
                 W4, b4.reshape(1, 1))
    return jnp.squeeze(out, axis=-1)
